# baseline (speedup 1.0000x reference)
import functools
import jax
import jax.numpy as jnp
from jax import lax
from jax.experimental import pallas as pl
from jax.experimental.pallas import tpu as pltpu
from jax.experimental.pallas import tpu_sc as plsc

BATCH = 16384
NUM_TYPES = 32
DIM = 128
NUM_CORES = 2
NUM_SUBCORES = 16
NUM_WORKERS = NUM_CORES * NUM_SUBCORES
ROWS_PER_W = BATCH // NUM_WORKERS
REP = 16
STEPS = ROWS_PER_W // REP


@functools.lru_cache(maxsize=1)
def _make_sc_broadcast():
    mesh = plsc.VectorSubcoreMesh(core_axis_name="c", subcore_axis_name="s")

    @functools.partial(
        pl.kernel,
        mesh=mesh,
        out_type=jax.ShapeDtypeStruct((BATCH, NUM_TYPES, DIM), jnp.float32),
        scratch_types=[
            pltpu.VMEM((REP, NUM_TYPES, DIM), jnp.float32),
            pltpu.SemaphoreType.DMA,
        ],
    )
    def _sc_broadcast(table_hbm, out_hbm, buf, sem):
        wid = lax.axis_index("s") * NUM_CORES + lax.axis_index("c")
        base = wid * ROWS_PER_W
        for r in range(REP):
            pltpu.sync_copy(table_hbm, buf.at[r])
        copies = []
        for j in range(STEPS):
            copies.append(
                pltpu.async_copy(buf, out_hbm.at[pl.ds(base + j * REP, REP)], sem)
            )
        for c in copies:
            c.wait()

    return _sc_broadcast


def kernel(action_mask, table):
    del action_mask
    return _make_sc_broadcast()(table)


# speedup vs baseline: 1.0015x; 1.0015x over previous
import functools
import jax
import jax.numpy as jnp
from jax import lax
from jax.experimental import pallas as pl
from jax.experimental.pallas import tpu as pltpu
from jax.experimental.pallas import tpu_sc as plsc

NUM_TYPES = 32
DIM = 128
REP = 16


@functools.lru_cache(maxsize=None)
def _make_sc_broadcast(batch):
    info = plsc.get_sparse_core_info()
    nc, ns = info.num_cores, info.num_subcores
    nw = nc * ns
    rows_per_w = batch // nw
    steps = rows_per_w // REP
    assert rows_per_w % REP == 0 and batch % nw == 0

    mesh = plsc.VectorSubcoreMesh(core_axis_name="c", subcore_axis_name="s")

    @functools.partial(
        pl.kernel,
        mesh=mesh,
        out_type=jax.ShapeDtypeStruct((batch, NUM_TYPES, DIM), jnp.float32),
        scratch_types=[
            pltpu.VMEM((REP, NUM_TYPES, DIM), jnp.float32),
            pltpu.SemaphoreType.DMA,
            pltpu.SemaphoreType.DMA,
        ],
    )
    def _sc_broadcast(table_hbm, out_hbm, buf, stage_sem, out_sem):
        wid = lax.axis_index("s") * nc + lax.axis_index("c")
        base = wid * rows_per_w
        stage = [
            pltpu.async_copy(table_hbm, buf.at[r], stage_sem) for r in range(REP)
        ]
        for c in stage:
            c.wait()
        copies = [
            pltpu.async_copy(buf, out_hbm.at[pl.ds(base + j * REP, REP)], out_sem)
            for j in range(steps)
        ]
        for c in copies:
            c.wait()

    return _sc_broadcast


def kernel(action_mask, table):
    batch = action_mask.shape[0]
    return _make_sc_broadcast(batch)(table)
